# baseline (speedup 1.0000x reference)
import functools

import jax
import jax.numpy as jnp
import numpy as np
from jax import lax
from jax.experimental import pallas as pl
from jax.experimental.pallas import tpu as pltpu
from jax.experimental.pallas import tpu_sc as plsc

BATCH = 16384
EMBED_K = 16
NUM_WORKERS = 32
BPW = BATCH // NUM_WORKERS
LANES = 16


@functools.partial(
    pl.kernel,
    out_type=jax.ShapeDtypeStruct((BATCH,), jnp.float32),
    mesh=plsc.VectorSubcoreMesh(core_axis_name="c", subcore_axis_name="s"),
    compiler_params=pltpu.CompilerParams(use_tc_tiling_on_sc=False),
    scratch_types=[
        pltpu.VMEM((BPW,), jnp.int32),
        pltpu.VMEM((BPW,), jnp.int32),
        pltpu.VMEM((BPW, EMBED_K), jnp.float32),
        pltpu.VMEM((BPW, EMBED_K), jnp.float32),
        pltpu.VMEM((BPW,), jnp.float32),
        pltpu.SemaphoreType.DMA,
        pltpu.SemaphoreType.DMA,
    ],
)
def _mf_predict(uidx_hbm, vidx_hbm, w_hbm, h_hbm, out_hbm,
                uidx_v, vidx_v, u_v, v_v, o_v, sem_u, sem_v):
    wid = lax.axis_index("s") * 2 + lax.axis_index("c")
    base = wid * BPW

    pltpu.sync_copy(uidx_hbm.at[pl.ds(base, BPW)], uidx_v)
    pltpu.sync_copy(vidx_hbm.at[pl.ds(base, BPW)], vidx_v)

    cu = pltpu.async_copy(w_hbm.at[uidx_v], u_v, sem_u)
    cv = pltpu.async_copy(h_hbm.at[vidx_v], v_v, sem_v)
    cu.wait()
    cv.wait()

    lane = lax.iota(jnp.int32, LANES)
    perms = [lane ^ sh for sh in (8, 4, 2, 1)]

    dnums = lax.GatherDimensionNumbers(
        offset_dims=(), collapsed_slice_dims=(0,), start_index_map=(0,))

    def shuffle(p, perm):
        return lax.gather(
            p, perm[:, None], dnums, slice_sizes=(1,),
            mode=lax.GatherScatterMode.PROMISE_IN_BOUNDS)

    def hsum(p):
        for perm in perms:
            p = p + shuffle(p, perm)
        return p

    def group_body(g, _):
        acc = jnp.zeros((LANES,), jnp.float32)
        for r in range(LANES):
            i = g * LANES + r
            s = hsum(u_v[i] * v_v[i])
            acc = jnp.where(lane == r, s, acc)
        o_v[pl.ds(g * LANES, LANES)] = 1.0 / (1.0 + jnp.exp(-acc))
        return 0

    lax.fori_loop(0, BPW // LANES, group_body, 0)

    pltpu.sync_copy(o_v, out_hbm.at[pl.ds(base, BPW)])


def kernel(x, W, H):
    uidx = x[:, 0].astype(jnp.int32)
    vidx = x[:, 1].astype(jnp.int32)
    return _mf_predict(uidx, vidx, W, H)


# speedup vs baseline: 1.0000x; 1.0000x over previous
import functools

import jax
import jax.numpy as jnp
from jax import lax
from jax.experimental import pallas as pl
from jax.experimental.pallas import tpu as pltpu
from jax.experimental.pallas import tpu_sc as plsc

BATCH = 16384
EMBED_K = 16
NUM_WORKERS = 32
BPW = BATCH // NUM_WORKERS
LANES = 16
GROUPS = BPW // LANES


@functools.partial(
    pl.kernel,
    out_type=jax.ShapeDtypeStruct((BATCH,), jnp.float32),
    mesh=plsc.VectorSubcoreMesh(core_axis_name="c", subcore_axis_name="s"),
    compiler_params=pltpu.CompilerParams(use_tc_tiling_on_sc=False),
    scratch_types=[
        pltpu.VMEM((BPW,), jnp.int32),
        pltpu.VMEM((BPW,), jnp.int32),
        pltpu.VMEM((BPW, EMBED_K), jnp.float32),
        pltpu.VMEM((BPW, EMBED_K), jnp.float32),
        pltpu.VMEM((BPW,), jnp.float32),
        pltpu.SemaphoreType.DMA,
        pltpu.SemaphoreType.DMA,
    ],
)
def _mf_predict(uidx_hbm, vidx_hbm, w_hbm, h_hbm, out_hbm,
                uidx_v, vidx_v, u_v, v_v, o_v, sem_u, sem_v):
    wid = lax.axis_index("s") * 2 + lax.axis_index("c")
    base = wid * BPW

    pltpu.sync_copy(uidx_hbm.at[pl.ds(base, BPW)], uidx_v)
    pltpu.sync_copy(vidx_hbm.at[pl.ds(base, BPW)], vidx_v)

    cu = pltpu.async_copy(w_hbm.at[uidx_v], u_v, sem_u)
    cv = pltpu.async_copy(h_hbm.at[vidx_v], v_v, sem_v)
    cu.wait()
    cv.wait()

    lane = lax.iota(jnp.int32, LANES)
    perms = [lane ^ sh for sh in (8, 4, 2, 1)]
    dnums = lax.GatherDimensionNumbers(
        offset_dims=(), collapsed_slice_dims=(0,), start_index_map=(0,))

    def shuffle(p, perm):
        return lax.gather(
            p, perm[:, None], dnums, slice_sizes=(1,),
            mode=lax.GatherScatterMode.PROMISE_IN_BOUNDS)

    def hsum(p):
        for perm in perms:
            p = p + shuffle(p, perm)
        return p

    def group_body(g, _):
        acc = jnp.zeros((LANES,), jnp.float32)
        for r in range(LANES):
            i = g * LANES + r
            s = hsum(u_v[i] * v_v[i])
            acc = jnp.where(lane == r, s, acc)
        o_v[pl.ds(g * LANES, LANES)] = 1.0 / (1.0 + jnp.exp(-acc))
        return 0

    lax.fori_loop(0, GROUPS, group_body, 0)

    pltpu.sync_copy(o_v, out_hbm.at[pl.ds(base, BPW)])


def kernel(x, W, H):
    uidx = x[:, 0].astype(jnp.int32)
    vidx = x[:, 1].astype(jnp.int32)
    return _mf_predict(uidx, vidx, W, H)
